# Initial kernel scaffold; baseline (speedup 1.0000x reference)
import jax
import jax.numpy as jnp
from jax.experimental import pallas as pl


def kernel(points, idx):
    raise NotImplementedError("write your pallas kernel here")


# speedup vs baseline: 15.1974x; 15.1974x over previous
import functools

import jax
import jax.numpy as jnp
from jax import lax
from jax.experimental import pallas as pl
from jax.experimental.pallas import tpu as pltpu
from jax.experimental.pallas import tpu_sc as plsc

_NUM_CORES = 2
_NUM_SUBCORES = 16
_NUM_WORKERS = _NUM_CORES * _NUM_SUBCORES
_CHUNK = 128


def _gather_body(rows_total, chunks_per_worker, C, table_hbm, idx_hbm, out_hbm,
                 idx_v, buf, sem0, sem1):
    wid = lax.axis_index("s") * _NUM_CORES + lax.axis_index("c")
    K = chunks_per_worker
    row_base = wid * (K * _CHUNK)
    idx_row_base = wid * K

    pltpu.sync_copy(idx_hbm.at[pl.ds(idx_row_base, K)], idx_v)

    sems = (sem0, sem1)

    def _start(j, slot, sem):
        pltpu.make_async_copy(table_hbm.at[idx_v.at[j]], buf.at[slot], sem).start()

    def _wait(j, slot, sem):
        pltpu.make_async_copy(table_hbm.at[idx_v.at[j]], buf.at[slot], sem).wait()

    _start(0, 0, sems[0])
    _start(1, 1, sems[1])

    def body(g, carry):
        for b in range(2):
            j = 2 * g + b
            _wait(j, b, sems[b])

            @pl.when(j + 2 < K)
            def _():
                _start(j + 2, b, sems[b])

            pltpu.sync_copy(buf.at[b],
                            out_hbm.at[pl.ds(row_base + j * _CHUNK, _CHUNK)])
        return carry

    lax.fori_loop(0, K // 2, body, 0)


def kernel(points, idx):
    B, N, C = points.shape
    Bi, M, S = idx.shape
    rows_total = B * M * S
    assert rows_total % (_NUM_WORKERS * _CHUNK) == 0
    chunks_per_worker = rows_total // (_NUM_WORKERS * _CHUNK)
    rows_per_worker = chunks_per_worker * _CHUNK
    rows_per_batch = M * S
    assert rows_per_batch % rows_per_worker == 0

    table = points.reshape(B * N, C)
    idx_flat = (idx.astype(jnp.int32).reshape(B, M * S)
                + (jnp.arange(B, dtype=jnp.int32) * N)[:, None])
    idx_flat = idx_flat.reshape(rows_total // _CHUNK, _CHUNK)

    mesh = plsc.VectorSubcoreMesh(core_axis_name="c", subcore_axis_name="s")
    run = pl.kernel(
        functools.partial(_gather_body, rows_total, chunks_per_worker, C),
        out_type=jax.ShapeDtypeStruct((rows_total, C), jnp.float32),
        mesh=mesh,
        scratch_types=[
            pltpu.VMEM((chunks_per_worker, _CHUNK), jnp.int32),
            pltpu.VMEM((2, _CHUNK, C), jnp.float32),
            pltpu.SemaphoreType.DMA,
            pltpu.SemaphoreType.DMA,
        ],
        compiler_params=pltpu.CompilerParams(use_tc_tiling_on_sc=False),
    )
    out = run(table, idx_flat)
    return out.reshape(B, M, S, C)


# speedup vs baseline: 15.3209x; 1.0081x over previous
import functools

import jax
import jax.numpy as jnp
from jax import lax
from jax.experimental import pallas as pl
from jax.experimental.pallas import tpu as pltpu
from jax.experimental.pallas import tpu_sc as plsc

_NUM_CORES = 2
_NUM_SUBCORES = 16
_NUM_WORKERS = _NUM_CORES * _NUM_SUBCORES
_CHUNK = 128
_GROUP = 4


def _gather_body(chunks_per_worker, C, table_hbm, idx_hbm, out_hbm,
                 idx_v, buf, gsem0, gsem1, wsem0, wsem1):
    wid = lax.axis_index("s") * _NUM_CORES + lax.axis_index("c")
    K = chunks_per_worker
    G = K // _GROUP
    rows_per_group = _GROUP * _CHUNK
    row_base = wid * (K * _CHUNK)
    idx_row_base = wid * K

    gsems = (gsem0, gsem1)
    wsems = (wsem0, wsem1)

    pltpu.sync_copy(idx_hbm.at[pl.ds(idx_row_base, K)], idx_v)

    def _gather_cp(g, q, slot):
        return pltpu.make_async_copy(
            table_hbm.at[idx_v.at[g * _GROUP + q]],
            buf.at[slot].at[pl.ds(q * _CHUNK, _CHUNK)],
            gsems[slot])

    def _write_cp(g, slot):
        return pltpu.make_async_copy(
            buf.at[slot],
            out_hbm.at[pl.ds(row_base + g * rows_per_group, rows_per_group)],
            wsems[slot])

    def _fire_group(g, slot):
        for q in range(_GROUP):
            _gather_cp(g, q, slot).start()

    def _drain_group(g, slot):
        for q in range(_GROUP):
            _gather_cp(g, q, slot).wait()

    _fire_group(0, 0)

    def body(g2, carry):
        for s in range(2):
            g = 2 * g2 + s
            nxt = 1 - s

            @pl.when((g >= 1) & (g + 1 < G))
            def _():
                _write_cp(g - 1, nxt).wait()

            @pl.when(g + 1 < G)
            def _():
                _fire_group(g + 1, nxt)

            _drain_group(g, s)
            _write_cp(g, s).start()
        return carry

    lax.fori_loop(0, G // 2, body, 0)

    _write_cp(G - 2, (G - 2) % 2).wait()
    _write_cp(G - 1, (G - 1) % 2).wait()


def kernel(points, idx):
    B, N, C = points.shape
    _, M, S = idx.shape
    rows_total = B * M * S
    assert rows_total % (_NUM_WORKERS * _CHUNK * _GROUP) == 0
    chunks_per_worker = rows_total // (_NUM_WORKERS * _CHUNK)
    rows_per_worker = chunks_per_worker * _CHUNK
    rows_per_batch = M * S
    assert rows_per_batch % rows_per_worker == 0
    assert (chunks_per_worker // _GROUP) % 2 == 0

    table = points.reshape(B * N, C)
    idx_flat = (idx.astype(jnp.int32).reshape(B, M * S)
                + (jnp.arange(B, dtype=jnp.int32) * N)[:, None])
    idx_flat = idx_flat.reshape(rows_total // _CHUNK, _CHUNK)

    mesh = plsc.VectorSubcoreMesh(core_axis_name="c", subcore_axis_name="s")
    run = pl.kernel(
        functools.partial(_gather_body, chunks_per_worker, C),
        out_type=jax.ShapeDtypeStruct((rows_total, C), jnp.float32),
        mesh=mesh,
        scratch_types=[
            pltpu.VMEM((chunks_per_worker, _CHUNK), jnp.int32),
            pltpu.VMEM((2, _GROUP * _CHUNK, C), jnp.float32),
            pltpu.SemaphoreType.DMA,
            pltpu.SemaphoreType.DMA,
            pltpu.SemaphoreType.DMA,
            pltpu.SemaphoreType.DMA,
        ],
        compiler_params=pltpu.CompilerParams(use_tc_tiling_on_sc=False),
    )
    out = run(table, idx_flat)
    return out.reshape(B, M, S, C)
